# baseline (speedup 1.0000x reference)
import jax
import jax.numpy as jnp
import numpy as np
from jax.experimental import pallas as pl

B = 1
N = 2048
D_MODEL = 1024
NUM_HEADS = 16
HEAD_DIM = D_MODEL // NUM_HEADS
QBLK = 256


def _qkv_kernel(xq_ref, xk_ref, xv_ref, wq_ref, wk_ref, wv_ref,
                bq_ref, bk_ref, bv_ref, q_ref, k_ref, v_ref):
    q_ref[...] = (jnp.dot(xq_ref[...], wq_ref[...],
                          preferred_element_type=jnp.float32)
                  + bq_ref[...]).astype(jnp.bfloat16)
    k_ref[...] = (jnp.dot(xk_ref[...], wk_ref[...],
                          preferred_element_type=jnp.float32)
                  + bk_ref[...]).astype(jnp.bfloat16)
    v_ref[...] = (jnp.dot(xv_ref[...], wv_ref[...],
                          preferred_element_type=jnp.float32)
                  + bv_ref[...]).astype(jnp.bfloat16)


def _attn_kernel(qlab_ref, vlab_ref, q_ref, k_ref, v_ref, o_ref):
    mask = qlab_ref[...] == vlab_ref[...]
    neg = jnp.where(mask, 0.0, -1e30)
    has = jnp.any(mask, axis=-1, keepdims=True).astype(jnp.float32)
    for h in range(NUM_HEADS):
        sl = slice(h * HEAD_DIM, (h + 1) * HEAD_DIM)
        q = q_ref[:, sl]
        k = k_ref[:, sl]
        s = jax.lax.dot_general(q, k, (((1,), (1,)), ((), ())),
                                preferred_element_type=jnp.float32) * 0.125
        masked = s + neg
        m = jnp.max(masked, axis=-1, keepdims=True)
        e = jnp.exp(masked - m)
        denom = jnp.sum(e, axis=-1, keepdims=True)
        p = (e / denom).astype(jnp.bfloat16)
        o = jnp.dot(p, v_ref[:, sl], preferred_element_type=jnp.float32)
        o_ref[:, sl] = (o * has).astype(jnp.bfloat16)


def _outproj_kernel(nv_ref, wm_ref, bm_ref, o_ref):
    o_ref[...] = jnp.dot(nv_ref[...], wm_ref[...],
                         preferred_element_type=jnp.float32) + bm_ref[...]


def _headmajor(W):
    return W.T.reshape(D_MODEL, HEAD_DIM, NUM_HEADS).transpose(0, 2, 1) \
              .reshape(D_MODEL, D_MODEL)


def _headmajor_b(b):
    return b.reshape(HEAD_DIM, NUM_HEADS).T.reshape(1, D_MODEL)


@jax.jit
def kernel(query, key, value, query_labels, value_labels,
           Wq, bq, Wk, bk, Wv, bv, Wm, bm):
    xq = query[0].T.astype(jnp.bfloat16)
    xk = key[0].T.astype(jnp.bfloat16)
    xv = value[0].T.astype(jnp.bfloat16)
    WqR, WkR, WvR = (_headmajor(Wq).astype(jnp.bfloat16),
                     _headmajor(Wk).astype(jnp.bfloat16),
                     _headmajor(Wv).astype(jnp.bfloat16))
    bqR, bkR, bvR = _headmajor_b(bq), _headmajor_b(bk), _headmajor_b(bv)
    WmRT = Wm.reshape(D_MODEL, HEAD_DIM, NUM_HEADS).transpose(0, 2, 1) \
             .reshape(D_MODEL, D_MODEL).T.astype(jnp.bfloat16)
    bmR = bm.reshape(1, D_MODEL)

    nblk = N // QBLK
    q2, k2, v2 = pl.pallas_call(
        _qkv_kernel,
        grid=(nblk,),
        in_specs=[
            pl.BlockSpec((QBLK, D_MODEL), lambda i: (i, 0)),
            pl.BlockSpec((QBLK, D_MODEL), lambda i: (i, 0)),
            pl.BlockSpec((QBLK, D_MODEL), lambda i: (i, 0)),
            pl.BlockSpec((D_MODEL, D_MODEL), lambda i: (0, 0)),
            pl.BlockSpec((D_MODEL, D_MODEL), lambda i: (0, 0)),
            pl.BlockSpec((D_MODEL, D_MODEL), lambda i: (0, 0)),
            pl.BlockSpec((1, D_MODEL), lambda i: (0, 0)),
            pl.BlockSpec((1, D_MODEL), lambda i: (0, 0)),
            pl.BlockSpec((1, D_MODEL), lambda i: (0, 0)),
        ],
        out_specs=[
            pl.BlockSpec((QBLK, D_MODEL), lambda i: (i, 0)),
            pl.BlockSpec((QBLK, D_MODEL), lambda i: (i, 0)),
            pl.BlockSpec((QBLK, D_MODEL), lambda i: (i, 0)),
        ],
        out_shape=[jax.ShapeDtypeStruct((N, D_MODEL), jnp.bfloat16)] * 3,
    )(xq, xk, xv, WqR, WkR, WvR, bqR, bkR, bvR)

    qlab = query_labels[0].reshape(N, 1)
    vlab = value_labels[0].reshape(1, N)
    attn = pl.pallas_call(
        _attn_kernel,
        grid=(nblk,),
        in_specs=[
            pl.BlockSpec((QBLK, 1), lambda i: (i, 0)),
            pl.BlockSpec((1, N), lambda i: (0, 0)),
            pl.BlockSpec((QBLK, D_MODEL), lambda i: (i, 0)),
            pl.BlockSpec((N, D_MODEL), lambda i: (0, 0)),
            pl.BlockSpec((N, D_MODEL), lambda i: (0, 0)),
        ],
        out_specs=pl.BlockSpec((QBLK, D_MODEL), lambda i: (i, 0)),
        out_shape=jax.ShapeDtypeStruct((N, D_MODEL), jnp.bfloat16),
    )(qlab, vlab, q2, k2, v2)

    outT = pl.pallas_call(
        _outproj_kernel,
        grid=(nblk,),
        in_specs=[
            pl.BlockSpec((QBLK, D_MODEL), lambda i: (i, 0)),
            pl.BlockSpec((D_MODEL, D_MODEL), lambda i: (0, 0)),
            pl.BlockSpec((1, D_MODEL), lambda i: (0, 0)),
        ],
        out_specs=pl.BlockSpec((QBLK, D_MODEL), lambda i: (i, 0)),
        out_shape=jax.ShapeDtypeStruct((N, D_MODEL), jnp.float32),
    )(attn, WmRT, bmR)

    return outT.T[None]


# speedup vs baseline: 1.3497x; 1.3497x over previous
import jax
import jax.numpy as jnp
import numpy as np
from jax.experimental import pallas as pl

B = 1
N = 2048
D_MODEL = 1024
NUM_HEADS = 16
HEAD_DIM = D_MODEL // NUM_HEADS
QBLK = 256

_QSCALE = 0.125 * 1.4426950408889634


def _qkv_kernel(xq_ref, xk_ref, xv_ref, wq_ref, wk_ref, wv_ref,
                bq_ref, bk_ref, bv_ref, q_ref, k_ref, v_ref):
    dims = (((0,), (0,)), ((), ()))
    xq = xq_ref[...].astype(jnp.bfloat16)
    xk = xk_ref[...].astype(jnp.bfloat16)
    xv = xv_ref[...].astype(jnp.bfloat16)
    q = jax.lax.dot_general(xq, wq_ref[...], dims,
                            preferred_element_type=jnp.float32) + bq_ref[...]
    q_ref[...] = (q * _QSCALE).astype(jnp.bfloat16)
    k = jax.lax.dot_general(xk, wk_ref[...], dims,
                            preferred_element_type=jnp.float32) + bk_ref[...]
    k_ref[...] = k.astype(jnp.bfloat16)
    v = jax.lax.dot_general(xv, wv_ref[...], dims,
                            preferred_element_type=jnp.float32) + bv_ref[...]
    v_ref[...] = v.astype(jnp.bfloat16)


def _attn_kernel(qlab_ref, vlab_ref, q_ref, k_ref, v_ref, o_ref):
    mask = qlab_ref[...] == vlab_ref[...]
    neg = jnp.where(mask, 0.0, -1e30)
    has = jnp.any(mask, axis=-1, keepdims=True).astype(jnp.float32)
    for h in range(NUM_HEADS):
        sl = slice(h * HEAD_DIM, (h + 1) * HEAD_DIM)
        q = q_ref[:, sl]
        k = k_ref[:, sl]
        s = jax.lax.dot_general(q, k, (((1,), (1,)), ((), ())),
                                preferred_element_type=jnp.float32)
        masked = s + neg
        m = jnp.max(masked, axis=-1, keepdims=True)
        e = jnp.exp2(masked - m).astype(jnp.bfloat16)
        denom = jnp.sum(e.astype(jnp.float32), axis=-1, keepdims=True)
        o = jnp.dot(e, v_ref[:, sl], preferred_element_type=jnp.float32)
        o_ref[:, sl] = (o * (has / denom)).astype(jnp.bfloat16)


def _outproj_kernel(nv_ref, wm_ref, bm_ref, o_ref):
    o_ref[...] = jax.lax.dot_general(
        wm_ref[...], nv_ref[...], (((1,), (1,)), ((), ())),
        preferred_element_type=jnp.float32) + bm_ref[...]


def _headmajor(W):
    return W.T.reshape(D_MODEL, HEAD_DIM, NUM_HEADS).transpose(0, 2, 1) \
              .reshape(D_MODEL, D_MODEL)


def _headmajor_b(b):
    return b.reshape(HEAD_DIM, NUM_HEADS).T.reshape(1, D_MODEL)


@jax.jit
def kernel(query, key, value, query_labels, value_labels,
           Wq, bq, Wk, bk, Wv, bv, Wm, bm):
    xq, xk, xv = query[0], key[0], value[0]
    WqR, WkR, WvR = (_headmajor(Wq).astype(jnp.bfloat16),
                     _headmajor(Wk).astype(jnp.bfloat16),
                     _headmajor(Wv).astype(jnp.bfloat16))
    bqR, bkR, bvR = _headmajor_b(bq), _headmajor_b(bk), _headmajor_b(bv)
    WmR = Wm.reshape(D_MODEL, HEAD_DIM, NUM_HEADS).transpose(0, 2, 1) \
            .reshape(D_MODEL, D_MODEL).astype(jnp.bfloat16)
    bmR = bm.reshape(D_MODEL, 1)

    nblk = N // QBLK
    q2, k2, v2 = pl.pallas_call(
        _qkv_kernel,
        grid=(nblk,),
        in_specs=[
            pl.BlockSpec((D_MODEL, QBLK), lambda i: (0, i)),
            pl.BlockSpec((D_MODEL, QBLK), lambda i: (0, i)),
            pl.BlockSpec((D_MODEL, QBLK), lambda i: (0, i)),
            pl.BlockSpec((D_MODEL, D_MODEL), lambda i: (0, 0)),
            pl.BlockSpec((D_MODEL, D_MODEL), lambda i: (0, 0)),
            pl.BlockSpec((D_MODEL, D_MODEL), lambda i: (0, 0)),
            pl.BlockSpec((1, D_MODEL), lambda i: (0, 0)),
            pl.BlockSpec((1, D_MODEL), lambda i: (0, 0)),
            pl.BlockSpec((1, D_MODEL), lambda i: (0, 0)),
        ],
        out_specs=[
            pl.BlockSpec((QBLK, D_MODEL), lambda i: (i, 0)),
            pl.BlockSpec((QBLK, D_MODEL), lambda i: (i, 0)),
            pl.BlockSpec((QBLK, D_MODEL), lambda i: (i, 0)),
        ],
        out_shape=[jax.ShapeDtypeStruct((N, D_MODEL), jnp.bfloat16)] * 3,
    )(xq, xk, xv, WqR, WkR, WvR, bqR, bkR, bvR)

    qlab = query_labels[0].reshape(N, 1)
    vlab = value_labels[0].reshape(1, N)
    attn = pl.pallas_call(
        _attn_kernel,
        grid=(nblk,),
        in_specs=[
            pl.BlockSpec((QBLK, 1), lambda i: (i, 0)),
            pl.BlockSpec((1, N), lambda i: (0, 0)),
            pl.BlockSpec((QBLK, D_MODEL), lambda i: (i, 0)),
            pl.BlockSpec((N, D_MODEL), lambda i: (0, 0)),
            pl.BlockSpec((N, D_MODEL), lambda i: (0, 0)),
        ],
        out_specs=pl.BlockSpec((QBLK, D_MODEL), lambda i: (i, 0)),
        out_shape=jax.ShapeDtypeStruct((N, D_MODEL), jnp.bfloat16),
    )(qlab, vlab, q2, k2, v2)

    out = pl.pallas_call(
        _outproj_kernel,
        grid=(nblk,),
        in_specs=[
            pl.BlockSpec((QBLK, D_MODEL), lambda i: (i, 0)),
            pl.BlockSpec((D_MODEL, D_MODEL), lambda i: (0, 0)),
            pl.BlockSpec((D_MODEL, 1), lambda i: (0, 0)),
        ],
        out_specs=pl.BlockSpec((D_MODEL, QBLK), lambda i: (0, i)),
        out_shape=jax.ShapeDtypeStruct((D_MODEL, N), jnp.float32),
    )(attn, WmR, bmR)

    return out[None]
